# baseline (speedup 1.0000x reference)
import functools

import jax
import jax.numpy as jnp
from jax import lax
from jax.experimental import pallas as pl
from jax.experimental.pallas import tpu as pltpu
from jax.experimental.pallas import tpu_sc as plsc

NUM_CLASSES = 100000
DIM = 128
BATCH = 16384
NC = 2
NS = 16
NW = NC * NS
ROWS_PER_W = BATCH // NW
SUB = 128
NSUB = ROWS_PER_W // SUB
HIST_PER_TILE = 6272
HIST_PAD = NS * HIST_PER_TILE
Y_PER_TILE = BATCH // NS

_mesh = plsc.VectorSubcoreMesh(core_axis_name="c", subcore_axis_name="s")


@functools.partial(
    pl.kernel,
    out_type=(jax.ShapeDtypeStruct((BATCH, DIM), jnp.float32),
              jax.ShapeDtypeStruct((NW, 1, ROWS_PER_W), jnp.float32)),
    mesh=_mesh,
    scratch_types=[
        pltpu.VMEM((HIST_PER_TILE,), jnp.float32),
        pltpu.VMEM((Y_PER_TILE,), jnp.float32),
        pltpu.VMEM((Y_PER_TILE,), jnp.int32),
        pltpu.VMEM((ROWS_PER_W,), jnp.int32),
        pltpu.VMEM((ROWS_PER_W,), jnp.float32),
        pltpu.VMEM((1, ROWS_PER_W), jnp.float32),
        pltpu.VMEM((ROWS_PER_W, DIM), jnp.float32),
        pltpu.VMEM_SHARED((HIST_PAD,), jnp.float32),
        pltpu.SemaphoreType.DMA,
        pltpu.SemaphoreType.DMA,
    ],
)
def _sc_stage(y_hbm, centers_hbm, gat_hbm, inv_hbm,
              zbuf, ones_v, ych, idx_v, cnt_v, inv_v, gbuf, hist, sem_g, sem_o):
    cid = lax.axis_index("c")
    sid = lax.axis_index("s")
    wid = cid * NS + sid
    base = wid * ROWS_PER_W

    zeros16 = jnp.zeros((16,), jnp.float32)
    ones16 = jnp.ones((16,), jnp.float32)

    pltpu.sync_copy(y_hbm.at[pl.ds(base, ROWS_PER_W)], idx_v)
    for t in range(NSUB):
        pltpu.async_copy(
            centers_hbm.at[idx_v.at[pl.ds(t * SUB, SUB)]],
            gbuf.at[pl.ds(t * SUB, SUB)], sem_g)

    with jax.named_scope("fills"):
        def fill_z(i, carry):
            zbuf[pl.ds(i * 16, 16)] = zeros16
            return carry

        lax.fori_loop(0, HIST_PER_TILE // 16, fill_z, 0, unroll=8)

        def fill_o(i, carry):
            ones_v[pl.ds(i * 16, 16)] = ones16
            return carry

        lax.fori_loop(0, Y_PER_TILE // 16, fill_o, 0, unroll=8)

    with jax.named_scope("hist"):
        pltpu.sync_copy(zbuf, hist.at[pl.ds(sid * HIST_PER_TILE, HIST_PER_TILE)])
        pltpu.sync_copy(y_hbm.at[pl.ds(sid * Y_PER_TILE, Y_PER_TILE)], ych)
        plsc.subcore_barrier()
        pltpu.sync_copy(ones_v, hist.at[ych], add=True)
        plsc.subcore_barrier()

    with jax.named_scope("counts"):
        pltpu.sync_copy(hist.at[idx_v], cnt_v)

        def fill_inv(i, carry):
            c16 = cnt_v[pl.ds(i * 16, 16)]
            inv_v[0, pl.ds(i * 16, 16)] = 0.5 / (c16 + 1.0)
            return carry

        lax.fori_loop(0, ROWS_PER_W // 16, fill_inv, 0, unroll=8)
        pltpu.sync_copy(inv_v, inv_hbm.at[wid])

    with jax.named_scope("drain"):
        for t in range(NSUB):
            pltpu.make_async_copy(
                centers_hbm.at[idx_v.at[pl.ds(t * SUB, SUB)]],
                gbuf.at[pl.ds(t * SUB, SUB)], sem_g).wait()
            pltpu.async_copy(gbuf.at[pl.ds(t * SUB, SUB)],
                             gat_hbm.at[pl.ds(base + t * SUB, SUB)], sem_o)
        for t in range(NSUB):
            pltpu.make_async_copy(gbuf.at[pl.ds(t * SUB, SUB)],
                                  gat_hbm.at[pl.ds(base + t * SUB, SUB)],
                                  sem_o).wait()


def _tc_body(h_ref, g_ref, iv_ref, acc_ref, o_ref):
    d = h_ref[...] - g_ref[...]
    ones_r = jnp.ones((1, DIM), jnp.float32)
    rs = lax.dot_general(ones_r, d * d, (((1,), (1,)), ((), ())),
                         preferred_element_type=jnp.float32)

    @pl.when(pl.program_id(0) == 0)
    def _():
        acc_ref[...] = jnp.zeros_like(acc_ref)

    acc_ref[...] += rs * iv_ref[0]

    @pl.when(pl.program_id(0) == _GRID - 1)
    def _():
        o_ref[0, 0] = jnp.sum(acc_ref[...])


_GRID = 64
_RB = BATCH // DIM // _GRID

_tc_loss = pl.pallas_call(
    _tc_body,
    grid=(_GRID,),
    in_specs=[
        pl.BlockSpec((_RB * DIM, DIM), lambda i: (i, 0)),
        pl.BlockSpec((_RB * DIM, DIM), lambda i: (i, 0)),
        pl.BlockSpec((1, 1, _RB * DIM), lambda i: (i // 2, 0, i % 2)),
    ],
    out_specs=[pl.BlockSpec((1, _RB * DIM), lambda i: (0, 0)),
               pl.BlockSpec(memory_space=pltpu.SMEM)],
    out_shape=[jax.ShapeDtypeStruct((1, _RB * DIM), jnp.float32),
               jax.ShapeDtypeStruct((1, 1), jnp.float32)],
)


def kernel(y, hidden, centers):
    gat, inv = _sc_stage(y.astype(jnp.int32), centers)
    _, out = _tc_loss(hidden, gat, inv)
    return out[0, 0]


# speedup vs baseline: 1.2965x; 1.2965x over previous
import functools

import jax
import jax.numpy as jnp
from jax import lax
from jax.experimental import pallas as pl
from jax.experimental.pallas import tpu as pltpu
from jax.experimental.pallas import tpu_sc as plsc

NUM_CLASSES = 100000
DIM = 128
BATCH = 16384
NC = 2
NS = 16
NW = NC * NS
ROWS_PER_W = BATCH // NW
SUB = 128
NSUB = ROWS_PER_W // SUB
HIST_PER_TILE = 6272
HIST_PAD = NS * HIST_PER_TILE
Y_PER_TILE = BATCH // NS

_mesh = plsc.VectorSubcoreMesh(core_axis_name="c", subcore_axis_name="s")


@functools.partial(
    pl.kernel,
    out_type=(jax.ShapeDtypeStruct((BATCH, DIM), jnp.float32),
              jax.ShapeDtypeStruct((NW, ROWS_PER_W), jnp.float32)),
    mesh=_mesh,
    scratch_types=[
        pltpu.VMEM((HIST_PER_TILE,), jnp.float32),
        pltpu.VMEM((Y_PER_TILE,), jnp.float32),
        pltpu.VMEM((Y_PER_TILE,), jnp.int32),
        pltpu.VMEM((ROWS_PER_W,), jnp.int32),
        pltpu.VMEM((ROWS_PER_W,), jnp.float32),
        pltpu.VMEM((ROWS_PER_W,), jnp.float32),
        pltpu.VMEM((ROWS_PER_W, DIM), jnp.float32),
        pltpu.VMEM_SHARED((HIST_PAD,), jnp.float32),
        pltpu.SemaphoreType.DMA,
    ],
)
def _sc_stage(y_hbm, centers_hbm, gat_hbm, inv_hbm,
              zbuf, ones_v, ych, idx_v, cnt_v, inv_v, gbuf, hist, sem_g):
    cid = lax.axis_index("c")
    sid = lax.axis_index("s")
    wid = cid * NS + sid
    base = wid * ROWS_PER_W

    zeros16 = jnp.zeros((16,), jnp.float32)
    ones16 = jnp.ones((16,), jnp.float32)

    pltpu.sync_copy(y_hbm.at[pl.ds(base, ROWS_PER_W)], idx_v)
    for t in range(NSUB):
        pltpu.async_copy(
            centers_hbm.at[idx_v.at[pl.ds(t * SUB, SUB)]],
            gbuf.at[pl.ds(t * SUB, SUB)], sem_g)

    with jax.named_scope("fills"):
        def fill_z(i, carry):
            zbuf[pl.ds(i * 16, 16)] = zeros16
            return carry

        lax.fori_loop(0, HIST_PER_TILE // 16, fill_z, 0, unroll=8)

        def fill_o(i, carry):
            ones_v[pl.ds(i * 16, 16)] = ones16
            return carry

        lax.fori_loop(0, Y_PER_TILE // 16, fill_o, 0, unroll=8)

    with jax.named_scope("hist"):
        pltpu.sync_copy(zbuf, hist.at[pl.ds(sid * HIST_PER_TILE, HIST_PER_TILE)])
        pltpu.sync_copy(y_hbm.at[pl.ds(sid * Y_PER_TILE, Y_PER_TILE)], ych)
        plsc.subcore_barrier()
        pltpu.sync_copy(ones_v, hist.at[ych], add=True)
        plsc.subcore_barrier()

    with jax.named_scope("counts"):
        pltpu.sync_copy(hist.at[idx_v], cnt_v)

        def fill_inv(i, carry):
            c16 = cnt_v[pl.ds(i * 16, 16)]
            inv_v[pl.ds(i * 16, 16)] = 0.5 / (c16 + 1.0)
            return carry

        lax.fori_loop(0, ROWS_PER_W // 16, fill_inv, 0, unroll=8)
        pltpu.sync_copy(inv_v, inv_hbm.at[wid])

    with jax.named_scope("drain"):
        for t in range(NSUB):
            pltpu.make_async_copy(
                centers_hbm.at[idx_v.at[pl.ds(t * SUB, SUB)]],
                gbuf.at[pl.ds(t * SUB, SUB)], sem_g).wait()
        pltpu.sync_copy(gbuf, gat_hbm.at[pl.ds(base, ROWS_PER_W)])


_GRID = 32
_RB = BATCH // DIM // _GRID


def _tc_body(h_ref, g_ref, iv_ref, acc_ref, o_ref):
    d = h_ref[...] - g_ref[...]
    w = d * d * iv_ref[0][:, :, None]
    p = w[0] + w[1] + w[2] + w[3]

    @pl.when(pl.program_id(0) == 0)
    def _():
        acc_ref[...] = jnp.zeros_like(acc_ref)

    acc_ref[...] += p

    @pl.when(pl.program_id(0) == _GRID - 1)
    def _():
        o_ref[0, 0] = jnp.sum(acc_ref[...])


_tc_loss = pl.pallas_call(
    _tc_body,
    grid=(_GRID,),
    in_specs=[
        pl.BlockSpec((_RB, DIM, DIM), lambda i: (i, 0, 0)),
        pl.BlockSpec((_RB, DIM, DIM), lambda i: (i, 0, 0)),
        pl.BlockSpec((1, _RB, DIM), lambda i: (i, 0, 0)),
    ],
    out_specs=[pl.BlockSpec((DIM, DIM), lambda i: (0, 0)),
               pl.BlockSpec(memory_space=pltpu.SMEM)],
    out_shape=[jax.ShapeDtypeStruct((DIM, DIM), jnp.float32),
               jax.ShapeDtypeStruct((1, 1), jnp.float32)],
)


def kernel(y, hidden, centers):
    gat, inv = _sc_stage(y.astype(jnp.int32), centers)
    h3 = hidden.reshape(BATCH // DIM, DIM, DIM)
    g3 = gat.reshape(BATCH // DIM, DIM, DIM)
    iv2 = inv.reshape(_GRID, _RB, DIM)
    _, out = _tc_loss(h3, g3, iv2)
    return out[0, 0]


# speedup vs baseline: 1.5402x; 1.1880x over previous
import functools

import jax
import jax.numpy as jnp
from jax import lax
from jax.experimental import pallas as pl
from jax.experimental.pallas import tpu as pltpu
from jax.experimental.pallas import tpu_sc as plsc

NUM_CLASSES = 100000
DIM = 128
BATCH = 16384
NC = 2
NS = 16
NW = NC * NS
ROWS_PER_W = BATCH // NW
SUB = 128
NSUB = ROWS_PER_W // SUB
HIST_PER_TILE = 6272
HIST_PAD = NS * HIST_PER_TILE
Y_PER_TILE = BATCH // NS

_mesh = plsc.VectorSubcoreMesh(core_axis_name="c", subcore_axis_name="s")


@functools.partial(
    pl.kernel,
    out_type=(jax.ShapeDtypeStruct((BATCH, DIM), jnp.float32),
              jax.ShapeDtypeStruct((NW, ROWS_PER_W), jnp.float32)),
    mesh=_mesh,
    scratch_types=[
        pltpu.VMEM((HIST_PER_TILE,), jnp.float32),
        pltpu.VMEM((Y_PER_TILE,), jnp.float32),
        pltpu.VMEM((Y_PER_TILE,), jnp.int32),
        pltpu.VMEM((ROWS_PER_W,), jnp.int32),
        pltpu.VMEM((ROWS_PER_W,), jnp.float32),
        pltpu.VMEM((ROWS_PER_W,), jnp.float32),
        pltpu.VMEM((ROWS_PER_W, DIM), jnp.float32),
        pltpu.VMEM_SHARED((HIST_PAD,), jnp.float32),
        pltpu.SemaphoreType.DMA,
    ],
)
def _sc_stage(y_hbm, centers_hbm, gat_hbm, inv_hbm,
              zbuf, ones_v, ych, idx_v, cnt_v, inv_v, gbuf, hist, sem_g):
    cid = lax.axis_index("c")
    sid = lax.axis_index("s")
    wid = cid * NS + sid
    base = wid * ROWS_PER_W

    zeros16 = jnp.zeros((16,), jnp.float32)
    ones16 = jnp.ones((16,), jnp.float32)

    pltpu.sync_copy(y_hbm.at[pl.ds(base, ROWS_PER_W)], idx_v)
    for t in range(NSUB):
        pltpu.async_copy(
            centers_hbm.at[idx_v.at[pl.ds(t * SUB, SUB)]],
            gbuf.at[pl.ds(t * SUB, SUB)], sem_g)

    with jax.named_scope("fills"):
        def fill_z(i, carry):
            zbuf[pl.ds(i * 16, 16)] = zeros16
            return carry

        lax.fori_loop(0, HIST_PER_TILE // 16, fill_z, 0, unroll=8)

        def fill_o(i, carry):
            ones_v[pl.ds(i * 16, 16)] = ones16
            return carry

        lax.fori_loop(0, Y_PER_TILE // 16, fill_o, 0, unroll=8)

    with jax.named_scope("hist"):
        pltpu.sync_copy(zbuf, hist.at[pl.ds(sid * HIST_PER_TILE, HIST_PER_TILE)])
        pltpu.sync_copy(y_hbm.at[pl.ds(sid * Y_PER_TILE, Y_PER_TILE)], ych)
        plsc.subcore_barrier()
        pltpu.sync_copy(ones_v, hist.at[ych], add=True)
        plsc.subcore_barrier()

    with jax.named_scope("counts"):
        pltpu.sync_copy(hist.at[idx_v], cnt_v)

        def fill_inv(i, carry):
            c16 = cnt_v[pl.ds(i * 16, 16)]
            inv_v[pl.ds(i * 16, 16)] = 0.5 / (c16 + 1.0)
            return carry

        lax.fori_loop(0, ROWS_PER_W // 16, fill_inv, 0, unroll=8)
        pltpu.sync_copy(inv_v, inv_hbm.at[wid])

    with jax.named_scope("drain"):
        for t in range(NSUB):
            pltpu.make_async_copy(
                centers_hbm.at[idx_v.at[pl.ds(t * SUB, SUB)]],
                gbuf.at[pl.ds(t * SUB, SUB)], sem_g).wait()
        pltpu.sync_copy(gbuf, gat_hbm.at[pl.ds(base, ROWS_PER_W)])


_GRID = 16
_RB = BATCH // DIM // _GRID


def _tc_body(h_ref, g_ref, iv_ref, acc_ref, o_ref):
    d = h_ref[...] - g_ref[...]
    w = d * d * iv_ref[0][:, :, None]
    p = ((w[0] + w[1]) + (w[2] + w[3])) + ((w[4] + w[5]) + (w[6] + w[7]))

    @pl.when(pl.program_id(0) == 0)
    def _():
        acc_ref[...] = jnp.zeros_like(acc_ref)

    acc_ref[...] += p

    @pl.when(pl.program_id(0) == _GRID - 1)
    def _():
        o_ref[0, 0] = jnp.sum(acc_ref[...])


_tc_loss = pl.pallas_call(
    _tc_body,
    grid=(_GRID,),
    in_specs=[
        pl.BlockSpec((_RB, DIM, DIM), lambda i: (i, 0, 0)),
        pl.BlockSpec((_RB, DIM, DIM), lambda i: (i, 0, 0)),
        pl.BlockSpec((1, _RB, DIM), lambda i: (i, 0, 0)),
    ],
    out_specs=[pl.BlockSpec((DIM, DIM), lambda i: (0, 0)),
               pl.BlockSpec(memory_space=pltpu.SMEM)],
    out_shape=[jax.ShapeDtypeStruct((DIM, DIM), jnp.float32),
               jax.ShapeDtypeStruct((1, 1), jnp.float32)],
)


def kernel(y, hidden, centers):
    gat, inv = _sc_stage(y.astype(jnp.int32), centers)
    h3 = hidden.reshape(BATCH // DIM, DIM, DIM)
    g3 = gat.reshape(BATCH // DIM, DIM, DIM)
    iv2 = inv.reshape(_GRID, _RB, DIM)
    _, out = _tc_loss(h3, g3, iv2)
    return out[0, 0]
